# baseline (speedup 1.0000x reference)
import jax
import jax.numpy as jnp
from jax.experimental import pallas as pl
from jax.experimental.pallas import tpu as pltpu

_TM = 512
_TK = 4096
_LANES = 128


def _vq_body(x_ref, e_ref, dist_ref, idx_ref, e2_ref, m_ref, ci_ref):
    i = pl.program_id(0)
    k = pl.program_id(1)
    nk = pl.num_programs(1)
    koff = k * _TK

    @pl.when(i == 0)
    def _():
        eb0 = e_ref[pl.ds(koff, _TK), :]
        e2_ref[0, pl.ds(koff, _TK)] = jnp.sum(eb0 * eb0, axis=1)

    xt = x_ref[...]
    x2 = jnp.sum(xt * xt, axis=1, keepdims=True)
    xs = xt * 2.0
    eb = e_ref[pl.ds(koff, _TK), :]
    mm2 = jax.lax.dot_general(
        xs, eb,
        dimension_numbers=(((1,), (1,)), ((), ())),
        preferred_element_type=jnp.float32)
    d = (x2 + e2_ref[:, pl.ds(koff, _TK)]) - mm2
    dist_ref[...] = d
    tm, tk = d.shape
    nchunk = tk // _LANES
    m = d[:, :_LANES]
    ci = jnp.zeros((tm, _LANES), dtype=jnp.int32)
    for c in range(1, nchunk):
        dc = d[:, c * _LANES:(c + 1) * _LANES]
        better = dc < m
        m = jnp.where(better, dc, m)
        ci = jnp.where(better, c, ci)

    @pl.when(k == 0)
    def _():
        m_ref[...] = m
        ci_ref[...] = ci

    @pl.when(k > 0)
    def _():
        better = m < m_ref[...]
        m_ref[...] = jnp.where(better, m, m_ref[...])
        ci_ref[...] = jnp.where(better, ci + k * nchunk, ci_ref[...])

    @pl.when(k == nk - 1)
    def _():
        mf = m_ref[...]
        cif = ci_ref[...]
        rowmin = jnp.min(mf, axis=1, keepdims=True)
        lane = jax.lax.broadcasted_iota(jnp.int32, (tm, _LANES), 1)
        gidx = cif * _LANES + lane
        idx_ref[...] = jnp.min(
            jnp.where(mf == rowmin, gidx, nk * tk), axis=1)


def kernel(x, embedding_weight):
    B, C, H, W = x.shape
    K, D = embedding_weight.shape
    M = B * H * W
    x_flat = jnp.transpose(x.reshape(B, C, H * W), (0, 2, 1))
    xm = x_flat.reshape(M, D)
    dist, idx = pl.pallas_call(
        _vq_body,
        grid=(M // _TM, K // _TK),
        in_specs=[
            pl.BlockSpec((_TM, D), lambda i, k: (i, 0)),
            pl.BlockSpec((K, D), lambda i, k: (0, 0)),
        ],
        out_specs=[
            pl.BlockSpec((_TM, _TK), lambda i, k: (i, k)),
            pl.BlockSpec((_TM,), lambda i, k: (i,)),
        ],
        out_shape=[
            jax.ShapeDtypeStruct((M, K), jnp.float32),
            jax.ShapeDtypeStruct((M,), jnp.int32),
        ],
        scratch_shapes=[
            pltpu.VMEM((1, K), jnp.float32),
            pltpu.VMEM((_TM, _LANES), jnp.float32),
            pltpu.VMEM((_TM, _LANES), jnp.int32),
        ],
    )(xm, embedding_weight)
    return (idx.reshape(B, H * W), dist.reshape(B, H * W, K))


# speedup vs baseline: 1.0326x; 1.0326x over previous
import jax
import jax.numpy as jnp
from jax.experimental import pallas as pl
from jax.experimental.pallas import tpu as pltpu

_TM = 512
_LANES = 128


def _vq_body(x_ref, e_ref, dist_ref, idx_ref, e2_ref):
    @pl.when(pl.program_id(0) == 0)
    def _():
        e2_ref[...] = jnp.sum(e_ref[...] ** 2, axis=1).reshape(1, -1)

    xc = x_ref[0]
    x2 = jnp.sum(xc * xc, axis=0, keepdims=True).T
    xs = xc * 2.0
    mm2 = jax.lax.dot_general(
        xs, e_ref[...],
        dimension_numbers=(((0,), (1,)), ((), ())),
        preferred_element_type=jnp.float32)
    d = (x2 + e2_ref[...]) - mm2
    dist_ref[...] = d
    tm, k = d.shape
    nchunk = k // _LANES
    m = d[:, :_LANES]
    ci = jnp.zeros((tm, _LANES), dtype=jnp.int32)
    for c in range(1, nchunk):
        dc = d[:, c * _LANES:(c + 1) * _LANES]
        better = dc < m
        m = jnp.where(better, dc, m)
        ci = jnp.where(better, c, ci)
    rowmin = jnp.min(m, axis=1, keepdims=True)
    lane = jax.lax.broadcasted_iota(jnp.int32, (tm, _LANES), 1)
    gidx = ci * _LANES + lane
    idx_ref[...] = jnp.min(jnp.where(m == rowmin, gidx, k), axis=1)


def kernel(x, embedding_weight):
    B, C, H, W = x.shape
    K, D = embedding_weight.shape
    M = B * H * W
    HW = H * W
    x3 = x.reshape(B, C, HW)
    nhw = HW // _TM
    dist, idx = pl.pallas_call(
        _vq_body,
        grid=(M // _TM,),
        in_specs=[
            pl.BlockSpec((1, C, _TM), lambda i: (i // nhw, 0, i % nhw)),
            pl.BlockSpec((K, D), lambda i: (0, 0)),
        ],
        out_specs=[
            pl.BlockSpec((_TM, K), lambda i: (i, 0)),
            pl.BlockSpec((_TM,), lambda i: (i,)),
        ],
        out_shape=[
            jax.ShapeDtypeStruct((M, K), jnp.float32),
            jax.ShapeDtypeStruct((M,), jnp.int32),
        ],
        scratch_shapes=[pltpu.VMEM((1, K), jnp.float32)],
    )(x3, embedding_weight)
    return (idx.reshape(B, H * W), dist.reshape(B, H * W, K))
